# baseline (speedup 1.0000x reference)
import functools
import jax
import jax.numpy as jnp
from jax import lax
from jax.experimental import pallas as pl
from jax.experimental.pallas import tpu as pltpu
from jax.experimental.pallas import tpu_sc as plsc

N = 10000
NP = 10240
E = 320000
D1 = 128
D2P = 48
NC = 2
NS = 16
NW = NC * NS
EW = 10240
EP = NW * EW
K = 128
NCH = EW // K
RPT = NP // NS


def _deg_body(col_hbm, ew_hbm, zeros_hbm, out_hbm, cidx, ewv, dacc, sem):
    cid = lax.axis_index("c")
    sid = lax.axis_index("s")
    wid = sid * NC + cid
    pltpu.sync_copy(zeros_hbm.at[pl.ds(sid * RPT, RPT)],
                    dacc.at[pl.ds(sid * RPT, RPT)])
    base0 = wid * EW

    def ld(i, carry):
        pltpu.async_copy(col_hbm.at[pl.ds(base0 + i * K, K)], cidx.at[i], sem)
        pltpu.async_copy(ew_hbm.at[pl.ds(base0 + i * K, K)], ewv.at[i], sem)
        return carry

    lax.fori_loop(0, NCH, ld, 0)

    def ld_drain(i, carry):
        pltpu.make_async_copy(col_hbm.at[pl.ds(base0 + i * K, K)], cidx.at[i], sem).wait()
        pltpu.make_async_copy(ew_hbm.at[pl.ds(base0 + i * K, K)], ewv.at[i], sem).wait()
        return carry

    lax.fori_loop(0, NCH, ld_drain, 0)
    plsc.subcore_barrier()

    def fire(i, carry):
        pltpu.async_copy(ewv.at[i], dacc.at[cidx.at[i]], sem, add=True)
        return carry

    lax.fori_loop(0, NCH, fire, 0)

    def drain(i, carry):
        pltpu.make_async_copy(ewv.at[i], dacc.at[cidx.at[i]], sem).wait()
        return carry

    lax.fori_loop(0, NCH, drain, 0)
    plsc.subcore_barrier()
    pltpu.sync_copy(dacc.at[pl.ds(sid * RPT, RPT)],
                    out_hbm.at[cid, pl.ds(sid * RPT, RPT)])


_deg_kernel = pl.kernel(
    _deg_body,
    out_type=jax.ShapeDtypeStruct((NC, NP), jnp.float32),
    mesh=plsc.VectorSubcoreMesh(core_axis_name="c", subcore_axis_name="s"),
    scratch_types=[
        pltpu.VMEM((NCH, K), jnp.int32),
        pltpu.VMEM((NCH, K), jnp.float32),
        pltpu.VMEM_SHARED((NP,), jnp.float32),
        pltpu.SemaphoreType.DMA,
    ],
)


NTR = N // NS


def _edge_body(D, NPH, row_hbm, col_hbm, ew_hbm, xs_hbm, zeros_hbm, out_hbm,
               ridx, cidx, ewv, rows0, rows1, rows2, rows3, table, acc,
               gsem0, gsem1, gsem2, gsem3, ssem0, ssem1, ssem2, ssem3,
               isem0, isem1, isem2, isem3):
    cid = lax.axis_index("c")
    sid = lax.axis_index("s")
    wid = sid * NC + cid
    rows = (rows0, rows1, rows2, rows3)
    gsem = (gsem0, gsem1, gsem2, gsem3)
    ssem = (ssem0, ssem1, ssem2, ssem3)
    isem = (isem0, isem1, isem2, isem3)
    base0 = wid * EW

    def fire_idx(i, slot):
        pltpu.async_copy(row_hbm.at[pl.ds(base0 + i * K, K)], ridx.at[slot], isem[slot])
        pltpu.async_copy(col_hbm.at[pl.ds(base0 + i * K, K)], cidx.at[i % 8], isem[slot])
        pltpu.async_copy(ew_hbm.at[pl.ds(base0 + i * K, K)], ewv.at[slot], isem[slot])

    def wait_idx(i, slot):
        pltpu.make_async_copy(row_hbm.at[pl.ds(base0 + i * K, K)], ridx.at[slot], isem[slot]).wait()
        pltpu.make_async_copy(col_hbm.at[pl.ds(base0 + i * K, K)], cidx.at[i % 8], isem[slot]).wait()
        pltpu.make_async_copy(ew_hbm.at[pl.ds(base0 + i * K, K)], ewv.at[slot], isem[slot]).wait()

    def scale(slot, buf):
        def grp_body(grp, c2):
            ewg = ewv[slot, pl.ds(grp * 16, 16)]
            for jj in range(16):
                j = grp * 16 + jj
                s = ewg[jj]
                for g in range(D // 16):
                    buf[j, pl.ds(g * 16, 16)] = buf[j, pl.ds(g * 16, 16)] * s
            return c2
        lax.fori_loop(0, K // 16, grp_body, 0)

    def step(i, slot):
        pltpu.make_async_copy(table.at[ridx.at[slot]], rows[slot], gsem[slot]).wait()
        scale(slot, rows[slot])
        pltpu.async_copy(rows[slot], acc.at[cidx.at[i % 8]], ssem[slot], add=True)

    def wait_scatter(i, slot):
        pltpu.make_async_copy(rows[slot], acc.at[cidx.at[i % 8]], ssem[slot]).wait()

    def fire_gather(slot):
        pltpu.async_copy(table.at[ridx.at[slot]], rows[slot], gsem[slot])

    def phase(h, carry):
        pltpu.sync_copy(xs_hbm.at[pl.ds(h * N + sid * NTR, NTR)],
                        table.at[pl.ds(sid * NTR, NTR)])
        pltpu.sync_copy(zeros_hbm.at[pl.ds(sid * RPT, RPT)],
                        acc.at[pl.ds(sid * RPT, RPT)])
        plsc.subcore_barrier()

        for s in range(4):
            fire_idx(s, s)
        for s in range(2):
            wait_idx(s, s)
            fire_gather(s)
        for b4 in range(4):
            step(b4, b4)
            fire_idx(4 + b4, b4)
            wait_idx(2 + b4, (b4 + 2) % 4)
            if b4 >= 2:
                wait_scatter(b4 - 2, (b4 + 2) % 4)
            fire_gather((b4 + 2) % 4)

        def quad(i4, c2):
            for b4 in range(4):
                i = 4 * i4 + b4
                step(i, b4)
                fire_idx(i + 4, b4)
                wait_idx(i + 2, (b4 + 2) % 4)
                wait_scatter(i - 2, (b4 + 2) % 4)
                fire_gather((b4 + 2) % 4)
            return c2

        lax.fori_loop(1, NCH // 4 - 1, quad, 0)
        for b4 in range(4):
            i = NCH - 4 + b4
            step(i, b4)
            if b4 < 2:
                wait_idx(i + 2, (b4 + 2) % 4)
                wait_scatter(i - 2, (b4 + 2) % 4)
                fire_gather((b4 + 2) % 4)
        for b4 in range(4):
            wait_scatter(NCH - 4 + b4, b4)

        plsc.subcore_barrier()
        pltpu.sync_copy(acc.at[pl.ds(sid * RPT, RPT)],
                        out_hbm.at[cid, pl.ds(sid * RPT, RPT), pl.ds(h * D, D)])
        plsc.subcore_barrier()
        return carry

    lax.fori_loop(0, NPH, phase, 0)


def _make_edge_kernel(D, NPH):
    return pl.kernel(
        functools.partial(_edge_body, D, NPH),
        out_type=jax.ShapeDtypeStruct((NC, NP, NPH * D), jnp.float32),
        mesh=plsc.VectorSubcoreMesh(core_axis_name="c", subcore_axis_name="s"),
        scratch_types=[
            pltpu.VMEM((4, K), jnp.int32),
            pltpu.VMEM((8, K), jnp.int32),
            pltpu.VMEM((4, K), jnp.float32),
            pltpu.VMEM((K, D), jnp.float32),
            pltpu.VMEM((K, D), jnp.float32),
            pltpu.VMEM((K, D), jnp.float32),
            pltpu.VMEM((K, D), jnp.float32),
            pltpu.VMEM_SHARED((N, D), jnp.float32),
            pltpu.VMEM_SHARED((NP, D), jnp.float32),
        ] + [pltpu.SemaphoreType.DMA] * 12,
        compiler_params=pltpu.CompilerParams(use_tc_tiling_on_sc=False),
    )


_edge_kernel_1 = _make_edge_kernel(64, 2)
_edge_kernel_2 = _make_edge_kernel(D2P, 1)


def _dense1_body(deg_ref, x_ref, w_ref, xs_ref, dinv_ref):
    deg = (deg_ref[0, :N] + deg_ref[1, :N])[:, None] + 1.0
    dinv = jax.lax.rsqrt(deg)
    dinv_ref[...] = dinv
    xw = jnp.dot(x_ref[...], w_ref[...], preferred_element_type=jnp.float32)
    xs_ref[...] = dinv * xw


def _dense2_body(acc_ref, xs_ref, dinv_ref, b_ref, w_ref, hs_ref):
    dinv = dinv_ref[...]
    acc = acc_ref[0, :N, :] + acc_ref[1, :N, :]
    h = jnp.maximum(dinv * (acc + xs_ref[...]) + b_ref[...], 0.0)
    hw = jnp.dot(h, w_ref[...], preferred_element_type=jnp.float32)
    hs_ref[...] = dinv * hw


def _dense3_body(acc_ref, hs_ref, dinv_ref, b_ref, out_ref):
    acc = acc_ref[0, :N, :D2P] + acc_ref[1, :N, :D2P]
    out_ref[...] = (dinv_ref[...] * (acc + hs_ref[...]) + b_ref[...])[:, :40]


def kernel(in_feat, edge_index, edge_weight, W1, b1, W2, b2):
    row = edge_index[0].astype(jnp.int32)
    col = edge_index[1].astype(jnp.int32)
    pad = EP - E
    row = jnp.concatenate([row, jnp.zeros((pad,), jnp.int32)])
    col = jnp.concatenate([col, jnp.zeros((pad,), jnp.int32)])
    ew = jnp.concatenate([edge_weight, jnp.zeros((pad,), jnp.float32)])

    zeros1 = jnp.zeros((NP, 64), jnp.float32)
    zeros2 = jnp.zeros((NP, D2P), jnp.float32)
    zerosn = jnp.zeros((NP,), jnp.float32)
    W2p = jnp.pad(W2, ((0, 0), (0, D2P - W2.shape[1])))
    b2p = jnp.pad(b2, (0, D2P - b2.shape[0]))

    deg2 = _deg_kernel(col, ew, zerosn)

    xs, dinv = pl.pallas_call(
        _dense1_body,
        out_shape=(
            jax.ShapeDtypeStruct((N, D1), jnp.float32),
            jax.ShapeDtypeStruct((N, 1), jnp.float32),
        ),
    )(deg2, in_feat, W1)

    xs_halves = jnp.concatenate([xs[:, :64], xs[:, 64:]], axis=0)
    acc1 = _edge_kernel_1(row, col, ew, xs_halves, zeros1)

    hs = pl.pallas_call(
        _dense2_body,
        out_shape=jax.ShapeDtypeStruct((N, D2P), jnp.float32),
    )(acc1, xs, dinv, b1[None, :], W2p)

    acc2 = _edge_kernel_2(row, col, ew, hs, zeros2)

    out = pl.pallas_call(
        _dense3_body,
        out_shape=jax.ShapeDtypeStruct((N, 40), jnp.float32),
    )(acc2, hs, dinv, b2p[None, :])
    return out


# speedup vs baseline: 1.2512x; 1.2512x over previous
import functools
import jax
import jax.numpy as jnp
from jax import lax
from jax.experimental import pallas as pl
from jax.experimental.pallas import tpu as pltpu
from jax.experimental.pallas import tpu_sc as plsc

N = 10000
NP = 10240
E = 320000
D1 = 128
D2P = 48
NC = 2
NS = 16
NW = NC * NS
EW = 10240
EP = NW * EW
K = 128
NCH = EW // K
RPT = NP // NS


def _deg_body(col_hbm, ew_hbm, zeros_hbm, out_hbm, cidx, ewv, dacc, sem):
    cid = lax.axis_index("c")
    sid = lax.axis_index("s")
    wid = sid * NC + cid
    pltpu.sync_copy(zeros_hbm.at[pl.ds(sid * RPT, RPT)],
                    dacc.at[pl.ds(sid * RPT, RPT)])
    base0 = wid * EW

    def ld(i, carry):
        pltpu.async_copy(col_hbm.at[pl.ds(base0 + i * K, K)], cidx.at[i], sem)
        pltpu.async_copy(ew_hbm.at[pl.ds(base0 + i * K, K)], ewv.at[i], sem)
        return carry

    lax.fori_loop(0, NCH, ld, 0)

    def ld_drain(i, carry):
        pltpu.make_async_copy(col_hbm.at[pl.ds(base0 + i * K, K)], cidx.at[i], sem).wait()
        pltpu.make_async_copy(ew_hbm.at[pl.ds(base0 + i * K, K)], ewv.at[i], sem).wait()
        return carry

    lax.fori_loop(0, NCH, ld_drain, 0)
    plsc.subcore_barrier()

    def fire(i, carry):
        pltpu.async_copy(ewv.at[i], dacc.at[cidx.at[i]], sem, add=True)
        return carry

    lax.fori_loop(0, NCH, fire, 0)

    def drain(i, carry):
        pltpu.make_async_copy(ewv.at[i], dacc.at[cidx.at[i]], sem).wait()
        return carry

    lax.fori_loop(0, NCH, drain, 0)
    plsc.subcore_barrier()
    pltpu.sync_copy(dacc.at[pl.ds(sid * RPT, RPT)],
                    out_hbm.at[cid, pl.ds(sid * RPT, RPT)])


_deg_kernel = pl.kernel(
    _deg_body,
    out_type=jax.ShapeDtypeStruct((NC, NP), jnp.float32),
    mesh=plsc.VectorSubcoreMesh(core_axis_name="c", subcore_axis_name="s"),
    scratch_types=[
        pltpu.VMEM((NCH, K), jnp.int32),
        pltpu.VMEM((NCH, K), jnp.float32),
        pltpu.VMEM_SHARED((NP,), jnp.float32),
        pltpu.SemaphoreType.DMA,
    ],
)


NTR = N // NS


def _edge_body(D, NPH, DT, row_hbm, col_hbm, ew_hbm, ewb_hbm, xs_hbm,
               zeros_hbm, out_hbm,
               ridx, cidx, ewv, ewvb, rows0, rows1, rows2, rows3, table, acc,
               gsem0, gsem1, gsem2, gsem3, ssem0, ssem1, ssem2, ssem3,
               isem0, isem1, isem2, isem3):
    cid = lax.axis_index("c")
    sid = lax.axis_index("s")
    wid = sid * NC + cid
    rows = (rows0, rows1, rows2, rows3)
    gsem = (gsem0, gsem1, gsem2, gsem3)
    ssem = (ssem0, ssem1, ssem2, ssem3)
    isem = (isem0, isem1, isem2, isem3)
    base0 = wid * EW

    BF = DT == jnp.bfloat16

    def fire_idx(i, slot):
        pltpu.async_copy(row_hbm.at[pl.ds(base0 + i * K, K)], ridx.at[slot], isem[slot])
        pltpu.async_copy(col_hbm.at[pl.ds(base0 + i * K, K)], cidx.at[i % 8], isem[slot])
        pltpu.async_copy(ew_hbm.at[pl.ds(base0 + i * K, K)], ewv.at[slot], isem[slot])

    def wait_idx(i, slot):
        pltpu.make_async_copy(row_hbm.at[pl.ds(base0 + i * K, K)], ridx.at[slot], isem[slot]).wait()
        pltpu.make_async_copy(col_hbm.at[pl.ds(base0 + i * K, K)], cidx.at[i % 8], isem[slot]).wait()
        pltpu.make_async_copy(ew_hbm.at[pl.ds(base0 + i * K, K)], ewv.at[slot], isem[slot]).wait()

    def scale(slot, buf):
        def grp_body(grp, c2):
            ewg = ewv[slot, pl.ds(grp * 16, 16)]
            for jj in range(16):
                j = grp * 16 + jj
                s = ewg[jj]
                if BF:
                    sv = jnp.full((16,), s, jnp.float32)
                    sb = plsc.pack(sv, sv, format=plsc.PackFormat.INTERLEAVED)
                    for g in range(D // 32):
                        buf[j, pl.ds(g * 32, 32)] = buf[j, pl.ds(g * 32, 32)] * sb
                else:
                    for g in range(D // 16):
                        buf[j, pl.ds(g * 16, 16)] = buf[j, pl.ds(g * 16, 16)] * s
            return c2
        lax.fori_loop(0, K // 16, grp_body, 0)

    def step(i, slot):
        pltpu.make_async_copy(table.at[ridx.at[slot]], rows[slot], gsem[slot]).wait()
        scale(slot, rows[slot])
        pltpu.async_copy(rows[slot], acc.at[cidx.at[i % 8]], ssem[slot], add=True)

    def wait_scatter(i, slot):
        pltpu.make_async_copy(rows[slot], acc.at[cidx.at[i % 8]], ssem[slot]).wait()

    def fire_gather(slot):
        pltpu.async_copy(table.at[ridx.at[slot]], rows[slot], gsem[slot])

    def phase(h, carry):
        pltpu.sync_copy(xs_hbm.at[pl.ds(h * N + sid * NTR, NTR)],
                        table.at[pl.ds(sid * NTR, NTR)])
        pltpu.sync_copy(zeros_hbm.at[pl.ds(sid * RPT, RPT)],
                        acc.at[pl.ds(sid * RPT, RPT)])
        plsc.subcore_barrier()

        for s in range(4):
            fire_idx(s, s)
        for s in range(2):
            wait_idx(s, s)
            fire_gather(s)
        for b4 in range(4):
            step(b4, b4)
            fire_idx(4 + b4, b4)
            wait_idx(2 + b4, (b4 + 2) % 4)
            if b4 >= 2:
                wait_scatter(b4 - 2, (b4 + 2) % 4)
            fire_gather((b4 + 2) % 4)

        def quad(i4, c2):
            for b4 in range(4):
                i = 4 * i4 + b4
                step(i, b4)
                fire_idx(i + 4, b4)
                wait_idx(i + 2, (b4 + 2) % 4)
                wait_scatter(i - 2, (b4 + 2) % 4)
                fire_gather((b4 + 2) % 4)
            return c2

        lax.fori_loop(1, NCH // 4 - 1, quad, 0)
        for b4 in range(4):
            i = NCH - 4 + b4
            step(i, b4)
            if b4 < 2:
                wait_idx(i + 2, (b4 + 2) % 4)
                wait_scatter(i - 2, (b4 + 2) % 4)
                fire_gather((b4 + 2) % 4)
        for b4 in range(4):
            wait_scatter(NCH - 4 + b4, b4)

        plsc.subcore_barrier()
        pltpu.sync_copy(acc.at[pl.ds(sid * RPT, RPT)],
                        out_hbm.at[cid, pl.ds(sid * RPT, RPT), pl.ds(h * D, D)])
        plsc.subcore_barrier()
        return carry

    lax.fori_loop(0, NPH, phase, 0)


def _make_edge_kernel(D, NPH, DT):
    return pl.kernel(
        functools.partial(_edge_body, D, NPH, DT),
        out_type=jax.ShapeDtypeStruct((NC, NP, NPH * D), DT),
        mesh=plsc.VectorSubcoreMesh(core_axis_name="c", subcore_axis_name="s"),
        scratch_types=[
            pltpu.VMEM((4, K), jnp.int32),
            pltpu.VMEM((8, K), jnp.int32),
            pltpu.VMEM((4, K), jnp.float32),
            pltpu.VMEM((4, K), jnp.bfloat16),
            pltpu.VMEM((K, D), DT),
            pltpu.VMEM((K, D), DT),
            pltpu.VMEM((K, D), DT),
            pltpu.VMEM((K, D), DT),
            pltpu.VMEM_SHARED((N, D), DT),
            pltpu.VMEM_SHARED((NP, D), DT),
        ] + [pltpu.SemaphoreType.DMA] * 12,
        compiler_params=pltpu.CompilerParams(use_tc_tiling_on_sc=False,
                                             needs_layout_passes=False),
    )


_edge_kernel_1 = _make_edge_kernel(128, 1, jnp.bfloat16)
_edge_kernel_2 = _make_edge_kernel(D2P, 1, jnp.float32)


def _dense1_body(deg_ref, x_ref, w_ref, xs_ref, xsb_ref, dinv_ref):
    deg = (deg_ref[0, :N] + deg_ref[1, :N])[:, None] + 1.0
    dinv = jax.lax.rsqrt(deg)
    dinv_ref[...] = dinv
    xw = jnp.dot(x_ref[...], w_ref[...], preferred_element_type=jnp.float32)
    xs = dinv * xw
    xs_ref[...] = xs
    xsb_ref[...] = xs.astype(jnp.bfloat16)


def _dense2_body(acc_ref, xs_ref, dinv_ref, b_ref, w_ref, hs_ref):
    dinv = dinv_ref[...]
    acc = acc_ref[0, :N, :].astype(jnp.float32) + acc_ref[1, :N, :].astype(jnp.float32)
    h = jnp.maximum(dinv * (acc + xs_ref[...]) + b_ref[...], 0.0)
    hw = jnp.dot(h, w_ref[...], preferred_element_type=jnp.float32)
    hs_ref[...] = dinv * hw


def _dense3_body(acc_ref, hs_ref, dinv_ref, b_ref, out_ref):
    acc = acc_ref[0, :N, :D2P] + acc_ref[1, :N, :D2P]
    out_ref[...] = (dinv_ref[...] * (acc + hs_ref[...]) + b_ref[...])[:, :40]


def kernel(in_feat, edge_index, edge_weight, W1, b1, W2, b2):
    row = edge_index[0].astype(jnp.int32)
    col = edge_index[1].astype(jnp.int32)
    pad = EP - E
    row = jnp.concatenate([row, jnp.zeros((pad,), jnp.int32)])
    col = jnp.concatenate([col, jnp.zeros((pad,), jnp.int32)])
    ew = jnp.concatenate([edge_weight, jnp.zeros((pad,), jnp.float32)])
    ewb = ew.astype(jnp.bfloat16)

    zeros1 = jnp.zeros((NP, D1), jnp.bfloat16)
    zeros2 = jnp.zeros((NP, D2P), jnp.float32)
    zerosn = jnp.zeros((NP,), jnp.float32)
    W2p = jnp.pad(W2, ((0, 0), (0, D2P - W2.shape[1])))
    b2p = jnp.pad(b2, (0, D2P - b2.shape[0]))

    deg2 = _deg_kernel(col, ew, zerosn)

    xs, xsb, dinv = pl.pallas_call(
        _dense1_body,
        out_shape=(
            jax.ShapeDtypeStruct((N, D1), jnp.float32),
            jax.ShapeDtypeStruct((N, D1), jnp.bfloat16),
            jax.ShapeDtypeStruct((N, 1), jnp.float32),
        ),
    )(deg2, in_feat, W1)

    acc1 = _edge_kernel_1(row, col, ew, ewb, xsb, zeros1)

    hs = pl.pallas_call(
        _dense2_body,
        out_shape=jax.ShapeDtypeStruct((N, D2P), jnp.float32),
    )(acc1, xs, dinv, b1[None, :], W2p)

    acc2 = _edge_kernel_2(row, col, ew, ewb, hs, zeros2)

    out = pl.pallas_call(
        _dense3_body,
        out_shape=jax.ShapeDtypeStruct((N, 40), jnp.float32),
    )(acc2, hs, dinv, b2p[None, :])
    return out


# speedup vs baseline: 1.3336x; 1.0658x over previous
import functools
import jax
import jax.numpy as jnp
from jax import lax
from jax.experimental import pallas as pl
from jax.experimental.pallas import tpu as pltpu
from jax.experimental.pallas import tpu_sc as plsc

N = 10000
NP = 10240
E = 320000
D1 = 128
D2P = 64
NC = 2
NS = 16
NW = NC * NS
EW = 10240
EP = NW * EW
K = 128
NCH = EW // K
RPT = NP // NS


def _deg_body(col_hbm, ew_hbm, zeros_hbm, out_hbm, cidx, ewv, dacc, sem):
    cid = lax.axis_index("c")
    sid = lax.axis_index("s")
    wid = sid * NC + cid
    pltpu.sync_copy(zeros_hbm.at[pl.ds(sid * RPT, RPT)],
                    dacc.at[pl.ds(sid * RPT, RPT)])
    base0 = wid * EW

    def ld(i, carry):
        pltpu.async_copy(col_hbm.at[pl.ds(base0 + i * K, K)], cidx.at[i], sem)
        pltpu.async_copy(ew_hbm.at[pl.ds(base0 + i * K, K)], ewv.at[i], sem)
        return carry

    lax.fori_loop(0, NCH, ld, 0)

    def ld_drain(i, carry):
        pltpu.make_async_copy(col_hbm.at[pl.ds(base0 + i * K, K)], cidx.at[i], sem).wait()
        pltpu.make_async_copy(ew_hbm.at[pl.ds(base0 + i * K, K)], ewv.at[i], sem).wait()
        return carry

    lax.fori_loop(0, NCH, ld_drain, 0)
    plsc.subcore_barrier()

    def fire(i, carry):
        pltpu.async_copy(ewv.at[i], dacc.at[cidx.at[i]], sem, add=True)
        return carry

    lax.fori_loop(0, NCH, fire, 0)

    def drain(i, carry):
        pltpu.make_async_copy(ewv.at[i], dacc.at[cidx.at[i]], sem).wait()
        return carry

    lax.fori_loop(0, NCH, drain, 0)
    plsc.subcore_barrier()
    pltpu.sync_copy(dacc.at[pl.ds(sid * RPT, RPT)],
                    out_hbm.at[cid, pl.ds(sid * RPT, RPT)])


_deg_kernel = pl.kernel(
    _deg_body,
    out_type=jax.ShapeDtypeStruct((NC, NP), jnp.float32),
    mesh=plsc.VectorSubcoreMesh(core_axis_name="c", subcore_axis_name="s"),
    scratch_types=[
        pltpu.VMEM((NCH, K), jnp.int32),
        pltpu.VMEM((NCH, K), jnp.float32),
        pltpu.VMEM_SHARED((NP,), jnp.float32),
        pltpu.SemaphoreType.DMA,
    ],
)


NTR = N // NS


def _edge_body(D, NPH, DT, row_hbm, col_hbm, ew_hbm, xs_hbm,
               zeros_hbm, out_hbm,
               ridx, cidx, ewv, rows0, rows1, rows2, rows3, table, acc,
               gsem0, gsem1, gsem2, gsem3, ssem0, ssem1, ssem2, ssem3,
               isem0, isem1, isem2, isem3):
    cid = lax.axis_index("c")
    sid = lax.axis_index("s")
    wid = sid * NC + cid
    rows = (rows0, rows1, rows2, rows3)
    gsem = (gsem0, gsem1, gsem2, gsem3)
    ssem = (ssem0, ssem1, ssem2, ssem3)
    isem = (isem0, isem1, isem2, isem3)
    base0 = wid * EW

    BF = DT == jnp.bfloat16

    def fire_idx(i, slot):
        pltpu.async_copy(row_hbm.at[pl.ds(base0 + i * K, K)], ridx.at[slot], isem[slot])
        pltpu.async_copy(col_hbm.at[pl.ds(base0 + i * K, K)], cidx.at[i % 8], isem[slot])
        pltpu.async_copy(ew_hbm.at[pl.ds(base0 + i * K, K)], ewv.at[slot], isem[slot])

    def wait_idx(i, slot):
        pltpu.make_async_copy(row_hbm.at[pl.ds(base0 + i * K, K)], ridx.at[slot], isem[slot]).wait()
        pltpu.make_async_copy(col_hbm.at[pl.ds(base0 + i * K, K)], cidx.at[i % 8], isem[slot]).wait()
        pltpu.make_async_copy(ew_hbm.at[pl.ds(base0 + i * K, K)], ewv.at[slot], isem[slot]).wait()

    def scale(slot, buf):
        def grp_body(grp, c2):
            ewg = ewv[slot, pl.ds(grp * 16, 16)]
            for jj in range(16):
                j = grp * 16 + jj
                s = ewg[jj]
                if BF:
                    sv = jnp.full((16,), s, jnp.float32)
                    sb = plsc.pack(sv, sv, format=plsc.PackFormat.INTERLEAVED)
                    for g in range(D // 32):
                        buf[j, pl.ds(g * 32, 32)] = buf[j, pl.ds(g * 32, 32)] * sb
                else:
                    for g in range(D // 16):
                        buf[j, pl.ds(g * 16, 16)] = buf[j, pl.ds(g * 16, 16)] * s
            return c2
        lax.fori_loop(0, K // 16, grp_body, 0)

    def step(i, slot):
        pltpu.make_async_copy(table.at[ridx.at[slot]], rows[slot], gsem[slot]).wait()
        scale(slot, rows[slot])
        pltpu.async_copy(rows[slot], acc.at[cidx.at[i % 8]], ssem[slot], add=True)

    def wait_scatter(i, slot):
        pltpu.make_async_copy(rows[slot], acc.at[cidx.at[i % 8]], ssem[slot]).wait()

    def fire_gather(slot):
        pltpu.async_copy(table.at[ridx.at[slot]], rows[slot], gsem[slot])

    def phase(h, carry):
        pltpu.sync_copy(xs_hbm.at[pl.ds(h * N + sid * NTR, NTR)],
                        table.at[pl.ds(sid * NTR, NTR)])
        pltpu.sync_copy(zeros_hbm.at[pl.ds(sid * RPT, RPT)],
                        acc.at[pl.ds(sid * RPT, RPT)])
        plsc.subcore_barrier()

        for s in range(4):
            fire_idx(s, s)
        for s in range(2):
            wait_idx(s, s)
            fire_gather(s)
        for b4 in range(4):
            step(b4, b4)
            fire_idx(4 + b4, b4)
            wait_idx(2 + b4, (b4 + 2) % 4)
            if b4 >= 2:
                wait_scatter(b4 - 2, (b4 + 2) % 4)
            fire_gather((b4 + 2) % 4)

        def quad(i4, c2):
            for b4 in range(4):
                i = 4 * i4 + b4
                step(i, b4)
                fire_idx(i + 4, b4)
                wait_idx(i + 2, (b4 + 2) % 4)
                wait_scatter(i - 2, (b4 + 2) % 4)
                fire_gather((b4 + 2) % 4)
            return c2

        lax.fori_loop(1, NCH // 4 - 1, quad, 0)
        for b4 in range(4):
            i = NCH - 4 + b4
            step(i, b4)
            if b4 < 2:
                wait_idx(i + 2, (b4 + 2) % 4)
                wait_scatter(i - 2, (b4 + 2) % 4)
                fire_gather((b4 + 2) % 4)
        for b4 in range(4):
            wait_scatter(NCH - 4 + b4, b4)

        plsc.subcore_barrier()
        pltpu.sync_copy(acc.at[pl.ds(sid * RPT, RPT)],
                        out_hbm.at[cid, pl.ds(sid * RPT, RPT), pl.ds(h * D, D)])
        plsc.subcore_barrier()
        return carry

    lax.fori_loop(0, NPH, phase, 0)


def _make_edge_kernel(D, NPH, DT):
    return pl.kernel(
        functools.partial(_edge_body, D, NPH, DT),
        out_type=jax.ShapeDtypeStruct((NC, NP, NPH * D), DT),
        mesh=plsc.VectorSubcoreMesh(core_axis_name="c", subcore_axis_name="s"),
        scratch_types=[
            pltpu.VMEM((4, K), jnp.int32),
            pltpu.VMEM((8, K), jnp.int32),
            pltpu.VMEM((4, K), jnp.float32),
            pltpu.VMEM((K, D), DT),
            pltpu.VMEM((K, D), DT),
            pltpu.VMEM((K, D), DT),
            pltpu.VMEM((K, D), DT),
            pltpu.VMEM_SHARED((N, D), DT),
            pltpu.VMEM_SHARED((NP, D), DT),
        ] + [pltpu.SemaphoreType.DMA] * 12,
        compiler_params=pltpu.CompilerParams(use_tc_tiling_on_sc=False,
                                             needs_layout_passes=False),
    )


_edge_kernel_1 = _make_edge_kernel(128, 1, jnp.bfloat16)
_edge_kernel_2 = _make_edge_kernel(D2P, 1, jnp.bfloat16)


def _dense1_body(deg_ref, x_ref, w_ref, xs_ref, xsb_ref, dinv_ref):
    deg = (deg_ref[0, :N] + deg_ref[1, :N])[:, None] + 1.0
    dinv = jax.lax.rsqrt(deg)
    dinv_ref[...] = dinv
    xw = jnp.dot(x_ref[...], w_ref[...], preferred_element_type=jnp.float32)
    xs = dinv * xw
    xs_ref[...] = xs
    xsb_ref[...] = xs.astype(jnp.bfloat16)


def _dense2_body(acc_ref, xs_ref, dinv_ref, b_ref, w_ref, hs_ref, hsb_ref):
    dinv = dinv_ref[...]
    acc = acc_ref[0, :N, :].astype(jnp.float32) + acc_ref[1, :N, :].astype(jnp.float32)
    h = jnp.maximum(dinv * (acc + xs_ref[...]) + b_ref[...], 0.0)
    hw = jnp.dot(h, w_ref[...], preferred_element_type=jnp.float32)
    hs = dinv * hw
    hs_ref[...] = hs
    hsb_ref[...] = hs.astype(jnp.bfloat16)


def _dense3_body(acc_ref, hs_ref, dinv_ref, b_ref, out_ref):
    acc = (acc_ref[0, :N, :].astype(jnp.float32)
           + acc_ref[1, :N, :].astype(jnp.float32))
    out_ref[...] = (dinv_ref[...] * (acc + hs_ref[...]) + b_ref[...])[:, :40]


def kernel(in_feat, edge_index, edge_weight, W1, b1, W2, b2):
    row = edge_index[0].astype(jnp.int32)
    col = edge_index[1].astype(jnp.int32)
    pad = EP - E
    row = jnp.concatenate([row, jnp.zeros((pad,), jnp.int32)])
    col = jnp.concatenate([col, jnp.zeros((pad,), jnp.int32)])
    ew = jnp.concatenate([edge_weight, jnp.zeros((pad,), jnp.float32)])

    zeros1 = jnp.zeros((NP, D1), jnp.bfloat16)
    zeros2 = jnp.zeros((NP, D2P), jnp.bfloat16)
    zerosn = jnp.zeros((NP,), jnp.float32)
    W2p = jnp.pad(W2, ((0, 0), (0, D2P - W2.shape[1])))
    b2p = jnp.pad(b2, (0, D2P - b2.shape[0]))

    deg2 = _deg_kernel(col, ew, zerosn)

    xs, xsb, dinv = pl.pallas_call(
        _dense1_body,
        out_shape=(
            jax.ShapeDtypeStruct((N, D1), jnp.float32),
            jax.ShapeDtypeStruct((N, D1), jnp.bfloat16),
            jax.ShapeDtypeStruct((N, 1), jnp.float32),
        ),
    )(deg2, in_feat, W1)

    acc1 = _edge_kernel_1(row, col, ew, xsb, zeros1)

    hs, hsb = pl.pallas_call(
        _dense2_body,
        out_shape=(
            jax.ShapeDtypeStruct((N, D2P), jnp.float32),
            jax.ShapeDtypeStruct((N, D2P), jnp.bfloat16),
        ),
    )(acc1, xs, dinv, b1[None, :], W2p)

    acc2 = _edge_kernel_2(row, col, ew, hsb, zeros2)

    out = pl.pallas_call(
        _dense3_body,
        out_shape=jax.ShapeDtypeStruct((N, 40), jnp.float32),
    )(acc2, hs, dinv, b2p[None, :])
    return out
